# baseline (speedup 1.0000x reference)
import functools

import jax
import jax.numpy as jnp
from jax import lax
from jax.experimental import pallas as pl
from jax.experimental.pallas import tpu as pltpu
from jax.experimental.pallas import tpu_sc as plsc

_N = 10000
_E = 320000
_H = 128
_OUT = 3

_NC = 2
_NS = 16
_NW = _NC * _NS
_EPW = _E // _NW
_CG = 80
_GRP = 5
_NCH = _EPW // _CG
_NGRP = _NCH // _GRP

_SPR = 624
_TAIL = _N - _NS * _SPR
_ST = 48
_NST = _SPR // _ST

_SCG = 40
_SNCH = _EPW // _SCG
_SNGRP = _SNCH // _GRP
_GGRP = 4


def _elu(v):
    return jnp.where(v > 0, v, jnp.exp(v) - 1.0)


def _ln(v, g, b):
    m = jnp.mean(v, axis=-1, keepdims=True)
    d = v - m
    var = jnp.mean(d * d, axis=-1, keepdims=True)
    return d * lax.rsqrt(var + 1e-5) * g + b


def _full(*shape):
    return pl.BlockSpec(shape, lambda i: tuple(0 for _ in shape))


def _rows(bm, bn):
    return pl.BlockSpec((bm, bn), lambda i: (i, 0))


def _pre_body(x_ref, a_ref, b_ref, g0_ref, g1_ref):
    x = x_ref[...]
    g0_ref[...] = jnp.dot(x, a_ref[...], preferred_element_type=jnp.float32)
    g1_ref[...] = jnp.dot(x, b_ref[...], preferred_element_type=jnp.float32)


def _tc_pre(x, A, B):
    bm = 2000
    return pl.pallas_call(
        _pre_body,
        grid=(_N // bm,),
        in_specs=[_rows(bm, _H), _full(_H, _H), _full(_H, _H)],
        out_specs=[_rows(bm, _H), _rows(bm, _H)],
        out_shape=[jax.ShapeDtypeStruct((_N, _H), jnp.float32)] * 2,
    )(x, A, B)


def _edge_body(xg0_ref, xg1_ref, ea_ref, c_ref, w1_ref, b0_ref, b1_ref,
               g_ref, bb_ref, o_ref):
    ea = ea_ref[...]
    t = (xg0_ref[...] + xg1_ref[...]
         + jnp.dot(ea, c_ref[...], preferred_element_type=jnp.float32)
         + b0_ref[...])
    t = _elu(t)
    u = jnp.dot(t, w1_ref[...], preferred_element_type=jnp.float32) + b1_ref[...]
    o_ref[...] = _ln(ea + u, g_ref[...], bb_ref[...])


def _tc_edge(xg0, xg1, ea, C, W1, b0, b1, g, bb):
    be = 4000
    e = ea.shape[0]
    return pl.pallas_call(
        _edge_body,
        grid=(e // be,),
        in_specs=[_rows(be, _H), _rows(be, _H), _rows(be, _H),
                  _full(_H, _H), _full(_H, _H),
                  _full(1, _H), _full(1, _H), _full(1, _H), _full(1, _H)],
        out_specs=_rows(be, _H),
        out_shape=jax.ShapeDtypeStruct((e, _H), jnp.float32),
    )(xg0, xg1, ea, C, W1, b0.reshape(1, _H), b1.reshape(1, _H),
      g.reshape(1, _H), bb.reshape(1, _H))


def _node_common(x, s0, s1, s2, s3, c, p_ref, q_ref, r_ref, b0_ref, b1_ref,
                 g_ref, bb_ref):
    cnt = jnp.maximum(c[:, 0:1], 1.0)
    agg = ((s0 + s1) + (s2 + s3)) / cnt
    t = (jnp.dot(x, p_ref[...], preferred_element_type=jnp.float32)
         + jnp.dot(agg, q_ref[...], preferred_element_type=jnp.float32)
         + b0_ref[...])
    t = _elu(t)
    u = jnp.dot(t, r_ref[...], preferred_element_type=jnp.float32) + b1_ref[...]
    return _ln(x + u, g_ref[...], bb_ref[...])


def _node0_body(x_ref, s0_ref, s1_ref, s2_ref, s3_ref, c_ref,
                p_ref, q_ref, r_ref,
                b0_ref, b1_ref, g_ref, bb_ref, a_ref, b_ref,
                xo_ref, g0_ref, g1_ref):
    xn = _node_common(x_ref[...], s0_ref[...], s1_ref[...], s2_ref[...],
                      s3_ref[...], c_ref[...],
                      p_ref, q_ref, r_ref, b0_ref, b1_ref,
                      g_ref, bb_ref)
    xo_ref[...] = xn
    g0_ref[...] = jnp.dot(xn, a_ref[...], preferred_element_type=jnp.float32)
    g1_ref[...] = jnp.dot(xn, b_ref[...], preferred_element_type=jnp.float32)


def _node1_body(x_ref, s0_ref, s1_ref, s2_ref, s3_ref, c_ref,
                p_ref, q_ref, r_ref,
                b0_ref, b1_ref, g_ref, bb_ref, ow_ref, ob_ref, o_ref):
    xn = _node_common(x_ref[...], s0_ref[...], s1_ref[...], s2_ref[...],
                      s3_ref[...], c_ref[...],
                      p_ref, q_ref, r_ref, b0_ref, b1_ref,
                      g_ref, bb_ref)
    o_ref[...] = _elu(jnp.dot(xn, ow_ref[...],
                              preferred_element_type=jnp.float32) + ob_ref[...])


def _tc_node0(x, sa, sb, c, P, Q, R, b0, b1, g, bb, A, B):
    bm = 2000
    return pl.pallas_call(
        _node0_body,
        grid=(_N // bm,),
        in_specs=[_rows(bm, _H)] * 6 +
                 [_full(_H, _H), _full(_H, _H), _full(_H, _H),
                  _full(1, _H), _full(1, _H), _full(1, _H), _full(1, _H),
                  _full(_H, _H), _full(_H, _H)],
        out_specs=[_rows(bm, _H)] * 3,
        out_shape=[jax.ShapeDtypeStruct((_N, _H), jnp.float32)] * 3,
    )(x, sa[0], sa[1], sb[0], sb[1], c, P, Q, R,
      b0.reshape(1, _H), b1.reshape(1, _H),
      g.reshape(1, _H), bb.reshape(1, _H), A, B)


def _tc_node1(x, sa, sb, c, P, Q, R, b0, b1, g, bb, ow_pad, ob_pad):
    bm = 2000
    return pl.pallas_call(
        _node1_body,
        grid=(_N // bm,),
        in_specs=[_rows(bm, _H)] * 6 +
                 [_full(_H, _H), _full(_H, _H), _full(_H, _H),
                  _full(1, _H), _full(1, _H), _full(1, _H), _full(1, _H),
                  _full(_H, _H), _full(1, _H)],
        out_specs=_rows(bm, _H),
        out_shape=jax.ShapeDtypeStruct((_N, _H), jnp.float32),
    )(x, sa[0], sa[1], sb[0], sb[1], c, P, Q, R,
      b0.reshape(1, _H), b1.reshape(1, _H),
      g.reshape(1, _H), bb.reshape(1, _H), ow_pad, ob_pad)


@functools.lru_cache(maxsize=None)
def _sc_gather_kernel(e):
    epw = e // _NW
    cg = 80 if epw % 80 == 0 else 40
    nch = epw // cg

    def body(g0_hbm, g1_hbm, ei0_hbm, ei1_hbm, o0_hbm, o1_hbm,
             idx0_v, idx1_v, r0_v, r1_v, gsem, wsem):
        wid = lax.axis_index("s") * _NC + lax.axis_index("c")
        base_w = wid * epw
        pltpu.sync_copy(ei0_hbm.at[pl.ds(base_w, epw)], idx0_v)
        pltpu.sync_copy(ei1_hbm.at[pl.ds(base_w, epw)], idx1_v)

        def chunk(k, b):
            off_l = k * cg
            return (pltpu.async_copy(g0_hbm.at[idx0_v.at[pl.ds(off_l, cg)]],
                                     r0_v.at[b], gsem),
                    pltpu.async_copy(g1_hbm.at[idx1_v.at[pl.ds(off_l, cg)]],
                                     r1_v.at[b], gsem))

        def drain(k, b, cps):
            off_g = base_w + k * cg
            cps[0].wait()
            cps[1].wait()
            return (pltpu.async_copy(r0_v.at[b], o0_hbm.at[pl.ds(off_g, cg)],
                                     wsem),
                    pltpu.async_copy(r1_v.at[b], o1_hbm.at[pl.ds(off_g, cg)],
                                     wsem))

        def group(j, carry):
            k0 = j * _GGRP
            cps = [chunk(k0 + b, b) for b in range(_GGRP)]
            wrs = [drain(k0 + b, b, cps[b]) for b in range(_GGRP)]
            for w0, w1 in wrs:
                w0.wait()
                w1.wait()
            return carry

        lax.fori_loop(0, nch // _GGRP, group, 0)
        for k in range(nch - nch % _GGRP, nch):
            cps = chunk(k, 0)
            w0, w1 = drain(k, 0, cps)
            w0.wait()
            w1.wait()

    return pl.kernel(
        body,
        mesh=plsc.VectorSubcoreMesh(core_axis_name="c", subcore_axis_name="s"),
        out_type=[jax.ShapeDtypeStruct((e, _H), jnp.float32)] * 2,
        scratch_types=[
            pltpu.VMEM((epw,), jnp.int32),
            pltpu.VMEM((epw,), jnp.int32),
            pltpu.VMEM((_GGRP, cg, _H), jnp.float32),
            pltpu.VMEM((_GGRP, cg, _H), jnp.float32),
            pltpu.SemaphoreType.DMA,
            pltpu.SemaphoreType.DMA,
        ],
    )


def _sc_gather(g0, g1, ei0, ei1):
    return _sc_gather_kernel(ei0.shape[0])(g0, g1, ei0, ei1)


@functools.lru_cache(maxsize=None)
def _sc_scatter_kernel(e):
    epw = e // _NW
    nsgrp = epw // _SCG // _GRP

    def body(ea_hbm, ei1_hbm, osum_hbm,
             idx_v, buf_v, stage_v, ssum_sh, gsem, isem, asem):
        cid = lax.axis_index("c")
        sid = lax.axis_index("s")
        wid = sid * _NC + cid
        base_w = wid * epw
        zero16 = jnp.zeros((16,), jnp.float32)

        def zrow(r, c):
            for jj in range(_H // 16):
                stage_v[r, pl.ds(jj * 16, 16)] = zero16
            return c
        lax.fori_loop(0, _ST, zrow, 0)

        for t in range(_NST):
            pltpu.sync_copy(stage_v,
                            ssum_sh.at[pl.ds(sid * _SPR + t * _ST, _ST)])

        @pl.when(sid == 0)
        def _zero_tail():
            pltpu.sync_copy(stage_v.at[pl.ds(0, _TAIL)],
                            ssum_sh.at[pl.ds(_NS * _SPR, _TAIL)])

        plsc.subcore_barrier()

        def group(j, carry):
            base_g = base_w + j * _GRP * _SCG
            ops = []
            for b in range(_GRP):
                off = base_g + b * _SCG
                ops.append((pltpu.async_copy(ei1_hbm.at[pl.ds(off, _SCG)],
                                             idx_v.at[b], isem),
                            pltpu.async_copy(ea_hbm.at[pl.ds(off, _SCG)],
                                             buf_v.at[b], gsem)))
            adds = []
            for b in range(_GRP):
                ops[b][0].wait()
                ops[b][1].wait()
                adds.append(pltpu.async_copy(buf_v.at[b],
                                             ssum_sh.at[idx_v.at[b]], asem,
                                             add=True))
            for a in adds:
                a.wait()
            return carry

        lax.fori_loop(0, nsgrp, group, 0)
        plsc.subcore_barrier()

        for t in range(_NST):
            row0 = sid * _SPR + t * _ST
            pltpu.sync_copy(ssum_sh.at[pl.ds(row0, _ST)], stage_v)
            pltpu.sync_copy(stage_v, osum_hbm.at[cid, pl.ds(row0, _ST)])

        @pl.when(sid == 0)
        def _write_tail():
            pltpu.sync_copy(ssum_sh.at[pl.ds(_NS * _SPR, _TAIL)],
                            stage_v.at[pl.ds(0, _TAIL)])
            pltpu.sync_copy(stage_v.at[pl.ds(0, _TAIL)],
                            osum_hbm.at[cid, pl.ds(_NS * _SPR, _TAIL)])

    return pl.kernel(
        body,
        mesh=plsc.VectorSubcoreMesh(core_axis_name="c", subcore_axis_name="s"),
        out_type=jax.ShapeDtypeStruct((_NC, _N, _H), jnp.float32),
        scratch_types=[
            pltpu.VMEM((_GRP, _SCG), jnp.int32),
            pltpu.VMEM((_GRP, _SCG, _H), jnp.float32),
            pltpu.VMEM((_ST, _H), jnp.float32),
            pltpu.VMEM_SHARED((_N, _H), jnp.float32),
            pltpu.SemaphoreType.DMA,
            pltpu.SemaphoreType.DMA,
            pltpu.SemaphoreType.DMA,
        ],
    )


def _sc_scatter(ea, ei1):
    return _sc_scatter_kernel(ea.shape[0])(ea, ei1)


_CST = 104


def _sc_count_body(ei1_hbm, ocnt_hbm, idx_v, ones_v, cstage_v, scnt_sh,
                   gsem, asem):
    cid = lax.axis_index("c")
    sid = lax.axis_index("s")
    wid = sid * _NC + cid
    base_w = wid * _EPW
    zero16 = jnp.zeros((16,), jnp.float32)
    one16 = jnp.full((16,), 1.0, jnp.float32)

    def zrow(r, c):
        for jj in range(_H // 16):
            cstage_v[r, pl.ds(jj * 16, 16)] = zero16
        return c
    lax.fori_loop(0, _CST, zrow, 0)

    def orow(r, c):
        for jj in range(_H // 16):
            ones_v[r, pl.ds(jj * 16, 16)] = one16
        return c
    lax.fori_loop(0, _CG, orow, 0)

    for t in range(_SPR // _CST):
        pltpu.sync_copy(cstage_v,
                        scnt_sh.at[pl.ds(sid * _SPR + t * _CST, _CST)])

    @pl.when(sid == 0)
    def _zero_tail():
        pltpu.sync_copy(cstage_v.at[pl.ds(0, _TAIL)],
                        scnt_sh.at[pl.ds(_NS * _SPR, _TAIL)])

    plsc.subcore_barrier()

    def group(j, carry):
        base_g = base_w + j * _GRP * _CG
        ics = []
        for b in range(_GRP):
            off = base_g + b * _CG
            ics.append(pltpu.async_copy(ei1_hbm.at[pl.ds(off, _CG)],
                                        idx_v.at[b], gsem))
        adds = []
        for b in range(_GRP):
            ics[b].wait()
            adds.append(pltpu.async_copy(ones_v, scnt_sh.at[idx_v.at[b]],
                                         asem, add=True))
        for a in adds:
            a.wait()
        return carry

    lax.fori_loop(0, _NGRP, group, 0)
    plsc.subcore_barrier()

    for t in range(_SPR // _CST):
        row0 = sid * _SPR + t * _CST
        pltpu.sync_copy(scnt_sh.at[pl.ds(row0, _CST)], cstage_v)
        pltpu.sync_copy(cstage_v, ocnt_hbm.at[cid, pl.ds(row0, _CST)])

    @pl.when(sid == 0)
    def _write_tail():
        pltpu.sync_copy(scnt_sh.at[pl.ds(_NS * _SPR, _TAIL)],
                        cstage_v.at[pl.ds(0, _TAIL)])
        pltpu.sync_copy(cstage_v.at[pl.ds(0, _TAIL)],
                        ocnt_hbm.at[cid, pl.ds(_NS * _SPR, _TAIL)])


@functools.lru_cache(maxsize=None)
def _sc_count_kernel():
    return pl.kernel(
        _sc_count_body,
        mesh=plsc.VectorSubcoreMesh(core_axis_name="c", subcore_axis_name="s"),
        out_type=jax.ShapeDtypeStruct((_NC, _N, _H), jnp.float32),
        scratch_types=[
            pltpu.VMEM((_GRP, _CG), jnp.int32),
            pltpu.VMEM((_CG, _H), jnp.float32),
            pltpu.VMEM((_CST, _H), jnp.float32),
            pltpu.VMEM_SHARED((_N, _H), jnp.float32),
            pltpu.SemaphoreType.DMA,
            pltpu.SemaphoreType.DMA,
        ],
    )


def _sc_count(ei1):
    return _sc_count_kernel()(ei1)


def _round(ea_h, ei0_h, ei1_h, g0, g1, C, W1, eb0, eb1, elg, elb):
    xg = [_sc_gather(g0, g1, ei0_h[h], ei1_h[h]) for h in range(2)]
    ea_h = [_tc_edge(xg[h][0], xg[h][1], ea_h[h], C, W1, eb0, eb1, elg, elb)
            for h in range(2)]
    sums = [_sc_scatter(ea_h[h], ei1_h[h]) for h in range(2)]
    return ea_h, sums


def kernel(x, edge_index, edge_attr, edge_indices, edge_indices_f2c, clusters,
           batches, positions, lengthscales,
           ew00, eb00, ew01, eb01, elng0, elnb0, nw00, nb00, nw01, nb01,
           nlng0, nlnb0,
           ew10, eb10, ew11, eb11, elng1, elnb1, nw10, nb10, nw11, nb11,
           nlng1, nlnb1, ow, ob):
    ei = edge_indices[0]
    ei0 = ei[0]
    ei1 = ei[1]
    A0, B0, C0 = ew00[:_H], ew00[_H:2 * _H], ew00[2 * _H:]
    A1, B1, C1 = ew10[:_H], ew10[_H:2 * _H], ew10[2 * _H:]
    P0, Q0 = nw00[:_H], nw00[_H:]
    P1, Q1 = nw10[:_H], nw10[_H:]
    ow_pad = jnp.pad(ow, ((0, 0), (0, _H - _OUT)))
    ob_pad = jnp.pad(ob, (0, _H - _OUT)).reshape(1, _H)

    ocnt = _sc_count(ei1)
    cnt = ocnt[0] + ocnt[1]

    eh = _E // 2
    ei0_h = (ei0[:eh], ei0[eh:])
    ei1_h = (ei1[:eh], ei1[eh:])
    ea_h = (edge_attr[:eh], edge_attr[eh:])

    g0, g1 = _tc_pre(x, A0, B0)
    ea_h, sums = _round(ea_h, ei0_h, ei1_h, g0, g1,
                        C0, ew01, eb00, eb01, elng0, elnb0)
    x, g0, g1 = _tc_node0(x, sums[0], sums[1], cnt,
                          P0, Q0, nw01, nb00, nb01, nlng0, nlnb0, A1, B1)
    ea_h, sums = _round(ea_h, ei0_h, ei1_h, g0, g1,
                        C1, ew11, eb10, eb11, elng1, elnb1)
    out_pad = _tc_node1(x, sums[0], sums[1], cnt,
                        P1, Q1, nw11, nb10, nb11, nlng1, nlnb1, ow_pad, ob_pad)
    return out_pad[:, :_OUT], ei


# speedup vs baseline: 1.0782x; 1.0782x over previous
import functools

import jax
import jax.numpy as jnp
from jax import lax
from jax.experimental import pallas as pl
from jax.experimental.pallas import tpu as pltpu
from jax.experimental.pallas import tpu_sc as plsc

_N = 10000
_E = 320000
_H = 128
_OUT = 3

_NC = 2
_NS = 16
_NW = _NC * _NS
_EPW = _E // _NW
_CG = 80
_GRP = 5
_NCH = _EPW // _CG
_NGRP = _NCH // _GRP

_SPR = 624
_TAIL = _N - _NS * _SPR
_ST = 48
_NST = _SPR // _ST

_SCG = 40
_SNCH = _EPW // _SCG
_SNGRP = _SNCH // _GRP
_GGRP = 4


def _elu(v):
    return jnp.where(v > 0, v, jnp.exp(v) - 1.0)


def _ln(v, g, b):
    m = jnp.mean(v, axis=-1, keepdims=True)
    d = v - m
    var = jnp.mean(d * d, axis=-1, keepdims=True)
    return d * lax.rsqrt(var + 1e-5) * g + b


def _full(*shape):
    return pl.BlockSpec(shape, lambda i: tuple(0 for _ in shape))


def _rows(bm, bn):
    return pl.BlockSpec((bm, bn), lambda i: (i, 0))


def _pre_body(x_ref, a_ref, b_ref, g0_ref, g1_ref):
    x = x_ref[...]
    g0_ref[...] = jnp.dot(x, a_ref[...], preferred_element_type=jnp.float32)
    g1_ref[...] = jnp.dot(x, b_ref[...], preferred_element_type=jnp.float32)


def _tc_pre(x, A, B):
    bm = 2000
    return pl.pallas_call(
        _pre_body,
        grid=(_N // bm,),
        in_specs=[_rows(bm, _H), _full(_H, _H), _full(_H, _H)],
        out_specs=[_rows(bm, _H), _rows(bm, _H)],
        out_shape=[jax.ShapeDtypeStruct((_N, _H), jnp.float32)] * 2,
    )(x, A, B)


def _edge_body(xg0_ref, xg1_ref, ea_ref, c_ref, w1_ref, b0_ref, b1_ref,
               g_ref, bb_ref, o_ref):
    ea = ea_ref[...]
    t = (xg0_ref[...] + xg1_ref[...]
         + jnp.dot(ea, c_ref[...], preferred_element_type=jnp.float32)
         + b0_ref[...])
    t = _elu(t)
    u = jnp.dot(t, w1_ref[...], preferred_element_type=jnp.float32) + b1_ref[...]
    o_ref[...] = _ln(ea + u, g_ref[...], bb_ref[...])


def _tc_edge(xg0, xg1, ea, C, W1, b0, b1, g, bb):
    be = 4000
    e = ea.shape[0]
    return pl.pallas_call(
        _edge_body,
        grid=(e // be,),
        in_specs=[_rows(be, _H), _rows(be, _H), _rows(be, _H),
                  _full(_H, _H), _full(_H, _H),
                  _full(1, _H), _full(1, _H), _full(1, _H), _full(1, _H)],
        out_specs=_rows(be, _H),
        out_shape=jax.ShapeDtypeStruct((e, _H), jnp.float32),
    )(xg0, xg1, ea, C, W1, b0.reshape(1, _H), b1.reshape(1, _H),
      g.reshape(1, _H), bb.reshape(1, _H))


def _node_common(x, s0, s1, c, p_ref, q_ref, r_ref, b0_ref, b1_ref,
                 g_ref, bb_ref):
    cnt = jnp.maximum(c[:, 0:1], 1.0)
    agg = (s0 + s1) / cnt
    t = (jnp.dot(x, p_ref[...], preferred_element_type=jnp.float32)
         + jnp.dot(agg, q_ref[...], preferred_element_type=jnp.float32)
         + b0_ref[...])
    t = _elu(t)
    u = jnp.dot(t, r_ref[...], preferred_element_type=jnp.float32) + b1_ref[...]
    return _ln(x + u, g_ref[...], bb_ref[...])


def _node0_body(x_ref, s0_ref, s1_ref, c_ref,
                p_ref, q_ref, r_ref,
                b0_ref, b1_ref, g_ref, bb_ref, a_ref, b_ref,
                xo_ref, g0_ref, g1_ref):
    xn = _node_common(x_ref[...], s0_ref[...], s1_ref[...], c_ref[...],
                      p_ref, q_ref, r_ref, b0_ref, b1_ref,
                      g_ref, bb_ref)
    xo_ref[...] = xn
    g0_ref[...] = jnp.dot(xn, a_ref[...], preferred_element_type=jnp.float32)
    g1_ref[...] = jnp.dot(xn, b_ref[...], preferred_element_type=jnp.float32)


def _node1_body(x_ref, s0_ref, s1_ref, c_ref,
                p_ref, q_ref, r_ref,
                b0_ref, b1_ref, g_ref, bb_ref, ow_ref, ob_ref, o_ref):
    xn = _node_common(x_ref[...], s0_ref[...], s1_ref[...], c_ref[...],
                      p_ref, q_ref, r_ref, b0_ref, b1_ref,
                      g_ref, bb_ref)
    o_ref[...] = _elu(jnp.dot(xn, ow_ref[...],
                              preferred_element_type=jnp.float32) + ob_ref[...])


def _tc_node0(x, sa, c, P, Q, R, b0, b1, g, bb, A, B):
    bm = 2000
    return pl.pallas_call(
        _node0_body,
        grid=(_N // bm,),
        in_specs=[_rows(bm, _H)] * 4 +
                 [_full(_H, _H), _full(_H, _H), _full(_H, _H),
                  _full(1, _H), _full(1, _H), _full(1, _H), _full(1, _H),
                  _full(_H, _H), _full(_H, _H)],
        out_specs=[_rows(bm, _H)] * 3,
        out_shape=[jax.ShapeDtypeStruct((_N, _H), jnp.float32)] * 3,
    )(x, sa[0], sa[1], c, P, Q, R,
      b0.reshape(1, _H), b1.reshape(1, _H),
      g.reshape(1, _H), bb.reshape(1, _H), A, B)


def _tc_node1(x, sa, c, P, Q, R, b0, b1, g, bb, ow_pad, ob_pad):
    bm = 2000
    return pl.pallas_call(
        _node1_body,
        grid=(_N // bm,),
        in_specs=[_rows(bm, _H)] * 4 +
                 [_full(_H, _H), _full(_H, _H), _full(_H, _H),
                  _full(1, _H), _full(1, _H), _full(1, _H), _full(1, _H),
                  _full(_H, _H), _full(1, _H)],
        out_specs=_rows(bm, _H),
        out_shape=jax.ShapeDtypeStruct((_N, _H), jnp.float32),
    )(x, sa[0], sa[1], c, P, Q, R,
      b0.reshape(1, _H), b1.reshape(1, _H),
      g.reshape(1, _H), bb.reshape(1, _H), ow_pad, ob_pad)


@functools.lru_cache(maxsize=None)
def _sc_gather_kernel(e):
    epw = e // _NW
    cg = 80 if epw % 80 == 0 else 40
    nch = epw // cg

    def body(g0_hbm, g1_hbm, ei0_hbm, ei1_hbm, o0_hbm, o1_hbm,
             idx0_v, idx1_v, r0_v, r1_v, gsem, ws0, ws1, ws2, ws3):
        wsems = [ws0, ws1, ws2, ws3]
        wid = lax.axis_index("s") * _NC + lax.axis_index("c")
        base_w = wid * epw
        pltpu.sync_copy(ei0_hbm.at[pl.ds(base_w, epw)], idx0_v)
        pltpu.sync_copy(ei1_hbm.at[pl.ds(base_w, epw)], idx1_v)

        def chunk(k, b):
            off_l = k * cg
            return (pltpu.async_copy(g0_hbm.at[idx0_v.at[pl.ds(off_l, cg)]],
                                     r0_v.at[b], gsem),
                    pltpu.async_copy(g1_hbm.at[idx1_v.at[pl.ds(off_l, cg)]],
                                     r1_v.at[b], gsem))

        def wdrain(b):
            pltpu.make_async_copy(r0_v.at[b], o0_hbm.at[pl.ds(0, cg)],
                                  wsems[b]).wait()
            pltpu.make_async_copy(r1_v.at[b], o1_hbm.at[pl.ds(0, cg)],
                                  wsems[b]).wait()

        def group(j, carry):
            k0 = j * _GGRP
            cps = []
            for b in range(_GGRP):
                @pl.when(j > 0)
                def _d():
                    wdrain(b)
                cps.append(chunk(k0 + b, b))
            for b in range(_GGRP):
                off_g = base_w + (k0 + b) * cg
                cps[b][0].wait()
                cps[b][1].wait()
                pltpu.async_copy(r0_v.at[b], o0_hbm.at[pl.ds(off_g, cg)],
                                 wsems[b])
                pltpu.async_copy(r1_v.at[b], o1_hbm.at[pl.ds(off_g, cg)],
                                 wsems[b])
            return carry

        lax.fori_loop(0, nch // _GGRP, group, 0)
        for b in range(_GGRP):
            wdrain(b)
        for k in range(nch - nch % _GGRP, nch):
            cps = chunk(k, 0)
            off_g = base_w + k * cg
            cps[0].wait()
            cps[1].wait()
            pltpu.async_copy(r0_v.at[0], o0_hbm.at[pl.ds(off_g, cg)], ws0)
            pltpu.async_copy(r1_v.at[0], o1_hbm.at[pl.ds(off_g, cg)], ws0)
            wdrain(0)

    return pl.kernel(
        body,
        mesh=plsc.VectorSubcoreMesh(core_axis_name="c", subcore_axis_name="s"),
        out_type=[jax.ShapeDtypeStruct((e, _H), jnp.float32)] * 2,
        scratch_types=[
            pltpu.VMEM((epw,), jnp.int32),
            pltpu.VMEM((epw,), jnp.int32),
            pltpu.VMEM((_GGRP, cg, _H), jnp.float32),
            pltpu.VMEM((_GGRP, cg, _H), jnp.float32),
            pltpu.SemaphoreType.DMA,
            pltpu.SemaphoreType.DMA,
            pltpu.SemaphoreType.DMA,
            pltpu.SemaphoreType.DMA,
            pltpu.SemaphoreType.DMA,
        ],
    )


def _sc_gather(g0, g1, ei0, ei1):
    return _sc_gather_kernel(ei0.shape[0])(g0, g1, ei0, ei1)


@functools.lru_cache(maxsize=None)
def _sc_scatter_kernel(e):
    epw = e // _NW
    nsgrp = epw // _SCG // _GRP

    def body(ea_hbm, ei1_hbm, osum_hbm,
             idx_v, buf_v, stage_v, ssum_sh, gsem, isem,
             as0, as1, as2, as3, as4):
        asems = [as0, as1, as2, as3, as4]
        cid = lax.axis_index("c")
        sid = lax.axis_index("s")
        wid = sid * _NC + cid
        base_w = wid * epw
        zero16 = jnp.zeros((16,), jnp.float32)

        def zrow(r, c):
            for jj in range(_H // 16):
                stage_v[r, pl.ds(jj * 16, 16)] = zero16
            return c
        lax.fori_loop(0, _ST, zrow, 0)

        for t in range(_NST):
            pltpu.sync_copy(stage_v,
                            ssum_sh.at[pl.ds(sid * _SPR + t * _ST, _ST)])

        @pl.when(sid == 0)
        def _zero_tail():
            pltpu.sync_copy(stage_v.at[pl.ds(0, _TAIL)],
                            ssum_sh.at[pl.ds(_NS * _SPR, _TAIL)])

        plsc.subcore_barrier()

        def group(j, carry):
            base_g = base_w + j * _GRP * _SCG
            ops = []
            for b in range(_GRP):
                @pl.when(j > 0)
                def _d():
                    pltpu.make_async_copy(buf_v.at[b],
                                          ssum_sh.at[idx_v.at[b]],
                                          asems[b]).wait()
                off = base_g + b * _SCG
                ops.append((pltpu.async_copy(ei1_hbm.at[pl.ds(off, _SCG)],
                                             idx_v.at[b], isem),
                            pltpu.async_copy(ea_hbm.at[pl.ds(off, _SCG)],
                                             buf_v.at[b], gsem)))
            for b in range(_GRP):
                ops[b][0].wait()
                ops[b][1].wait()
                pltpu.async_copy(buf_v.at[b], ssum_sh.at[idx_v.at[b]],
                                 asems[b], add=True)
            return carry

        lax.fori_loop(0, nsgrp, group, 0)
        for b in range(_GRP):
            pltpu.make_async_copy(buf_v.at[b], ssum_sh.at[idx_v.at[b]],
                                  asems[b]).wait()
        plsc.subcore_barrier()

        for t in range(_NST):
            row0 = sid * _SPR + t * _ST
            pltpu.sync_copy(ssum_sh.at[pl.ds(row0, _ST)], stage_v)
            pltpu.sync_copy(stage_v, osum_hbm.at[cid, pl.ds(row0, _ST)])

        @pl.when(sid == 0)
        def _write_tail():
            pltpu.sync_copy(ssum_sh.at[pl.ds(_NS * _SPR, _TAIL)],
                            stage_v.at[pl.ds(0, _TAIL)])
            pltpu.sync_copy(stage_v.at[pl.ds(0, _TAIL)],
                            osum_hbm.at[cid, pl.ds(_NS * _SPR, _TAIL)])

    return pl.kernel(
        body,
        mesh=plsc.VectorSubcoreMesh(core_axis_name="c", subcore_axis_name="s"),
        out_type=jax.ShapeDtypeStruct((_NC, _N, _H), jnp.float32),
        scratch_types=[
            pltpu.VMEM((_GRP, _SCG), jnp.int32),
            pltpu.VMEM((_GRP, _SCG, _H), jnp.float32),
            pltpu.VMEM((_ST, _H), jnp.float32),
            pltpu.VMEM_SHARED((_N, _H), jnp.float32),
            pltpu.SemaphoreType.DMA,
            pltpu.SemaphoreType.DMA,
            pltpu.SemaphoreType.DMA,
            pltpu.SemaphoreType.DMA,
            pltpu.SemaphoreType.DMA,
            pltpu.SemaphoreType.DMA,
            pltpu.SemaphoreType.DMA,
        ],
    )


def _sc_scatter(ea, ei1):
    return _sc_scatter_kernel(ea.shape[0])(ea, ei1)


_CST = 104


def _sc_count_body(ei1_hbm, ocnt_hbm, idx_v, ones_v, cstage_v, scnt_sh,
                   gsem, asem):
    cid = lax.axis_index("c")
    sid = lax.axis_index("s")
    wid = sid * _NC + cid
    base_w = wid * _EPW
    zero16 = jnp.zeros((16,), jnp.float32)
    one16 = jnp.full((16,), 1.0, jnp.float32)

    def zrow(r, c):
        for jj in range(_H // 16):
            cstage_v[r, pl.ds(jj * 16, 16)] = zero16
        return c
    lax.fori_loop(0, _CST, zrow, 0)

    def orow(r, c):
        for jj in range(_H // 16):
            ones_v[r, pl.ds(jj * 16, 16)] = one16
        return c
    lax.fori_loop(0, _CG, orow, 0)

    for t in range(_SPR // _CST):
        pltpu.sync_copy(cstage_v,
                        scnt_sh.at[pl.ds(sid * _SPR + t * _CST, _CST)])

    @pl.when(sid == 0)
    def _zero_tail():
        pltpu.sync_copy(cstage_v.at[pl.ds(0, _TAIL)],
                        scnt_sh.at[pl.ds(_NS * _SPR, _TAIL)])

    plsc.subcore_barrier()

    def group(j, carry):
        base_g = base_w + j * _GRP * _CG
        ics = []
        for b in range(_GRP):
            off = base_g + b * _CG
            ics.append(pltpu.async_copy(ei1_hbm.at[pl.ds(off, _CG)],
                                        idx_v.at[b], gsem))
        adds = []
        for b in range(_GRP):
            ics[b].wait()
            adds.append(pltpu.async_copy(ones_v, scnt_sh.at[idx_v.at[b]],
                                         asem, add=True))
        for a in adds:
            a.wait()
        return carry

    lax.fori_loop(0, _NGRP, group, 0)
    plsc.subcore_barrier()

    for t in range(_SPR // _CST):
        row0 = sid * _SPR + t * _CST
        pltpu.sync_copy(scnt_sh.at[pl.ds(row0, _CST)], cstage_v)
        pltpu.sync_copy(cstage_v, ocnt_hbm.at[cid, pl.ds(row0, _CST)])

    @pl.when(sid == 0)
    def _write_tail():
        pltpu.sync_copy(scnt_sh.at[pl.ds(_NS * _SPR, _TAIL)],
                        cstage_v.at[pl.ds(0, _TAIL)])
        pltpu.sync_copy(cstage_v.at[pl.ds(0, _TAIL)],
                        ocnt_hbm.at[cid, pl.ds(_NS * _SPR, _TAIL)])


@functools.lru_cache(maxsize=None)
def _sc_count_kernel():
    return pl.kernel(
        _sc_count_body,
        mesh=plsc.VectorSubcoreMesh(core_axis_name="c", subcore_axis_name="s"),
        out_type=jax.ShapeDtypeStruct((_NC, _N, _H), jnp.float32),
        scratch_types=[
            pltpu.VMEM((_GRP, _CG), jnp.int32),
            pltpu.VMEM((_CG, _H), jnp.float32),
            pltpu.VMEM((_CST, _H), jnp.float32),
            pltpu.VMEM_SHARED((_N, _H), jnp.float32),
            pltpu.SemaphoreType.DMA,
            pltpu.SemaphoreType.DMA,
        ],
    )


def _sc_count(ei1):
    return _sc_count_kernel()(ei1)


def _round(ea, ei0, ei1, g0, g1, C, W1, eb0, eb1, elg, elb):
    xg0, xg1 = _sc_gather(g0, g1, ei0, ei1)
    ea = _tc_edge(xg0, xg1, ea, C, W1, eb0, eb1, elg, elb)
    sums = _sc_scatter(ea, ei1)
    return ea, sums


def kernel(x, edge_index, edge_attr, edge_indices, edge_indices_f2c, clusters,
           batches, positions, lengthscales,
           ew00, eb00, ew01, eb01, elng0, elnb0, nw00, nb00, nw01, nb01,
           nlng0, nlnb0,
           ew10, eb10, ew11, eb11, elng1, elnb1, nw10, nb10, nw11, nb11,
           nlng1, nlnb1, ow, ob):
    ei = edge_indices[0]
    ei0 = ei[0]
    ei1 = ei[1]
    A0, B0, C0 = ew00[:_H], ew00[_H:2 * _H], ew00[2 * _H:]
    A1, B1, C1 = ew10[:_H], ew10[_H:2 * _H], ew10[2 * _H:]
    P0, Q0 = nw00[:_H], nw00[_H:]
    P1, Q1 = nw10[:_H], nw10[_H:]
    ow_pad = jnp.pad(ow, ((0, 0), (0, _H - _OUT)))
    ob_pad = jnp.pad(ob, (0, _H - _OUT)).reshape(1, _H)

    ocnt = _sc_count(ei1)
    cnt = ocnt[0] + ocnt[1]

    g0, g1 = _tc_pre(x, A0, B0)
    ea, sums_a = _round(edge_attr, ei0, ei1, g0, g1,
                        C0, ew01, eb00, eb01, elng0, elnb0)
    x, g0, g1 = _tc_node0(x, sums_a, cnt,
                          P0, Q0, nw01, nb00, nb01, nlng0, nlnb0, A1, B1)
    ea, sums_b = _round(ea, ei0, ei1, g0, g1,
                        C1, ew11, eb10, eb11, elng1, elnb1)
    out_pad = _tc_node1(x, sums_b, cnt,
                        P1, Q1, nw11, nb10, nb11, nlng1, nlnb1, ow_pad, ob_pad)
    return out_pad[:, :_OUT], ei


# speedup vs baseline: 1.0957x; 1.0162x over previous
import functools

import jax
import jax.numpy as jnp
from jax import lax
from jax.experimental import pallas as pl
from jax.experimental.pallas import tpu as pltpu
from jax.experimental.pallas import tpu_sc as plsc

_N = 10000
_E = 320000
_H = 128
_OUT = 3

_NC = 2
_NS = 16
_NW = _NC * _NS
_EPW = _E // _NW
_CG = 80
_GRP = 5
_NCH = _EPW // _CG
_NGRP = _NCH // _GRP

_SPR = 624
_TAIL = _N - _NS * _SPR
_ST = 48
_NST = _SPR // _ST

_SCG = 80
_SGRP = 4
_GGRP = 4


def _elu(v):
    return jnp.where(v > 0, v, jnp.exp(v) - 1.0)


def _ln(v, g, b):
    m = jnp.mean(v, axis=-1, keepdims=True)
    d = v - m
    var = jnp.mean(d * d, axis=-1, keepdims=True)
    return d * lax.rsqrt(var + 1e-5) * g + b


def _full(*shape):
    return pl.BlockSpec(shape, lambda i: tuple(0 for _ in shape))


def _rows(bm, bn):
    return pl.BlockSpec((bm, bn), lambda i: (i, 0))


def _pre_body(x_ref, a_ref, b_ref, g0_ref, g1_ref):
    x = x_ref[...]
    g0_ref[...] = jnp.dot(x, a_ref[...], preferred_element_type=jnp.float32)
    g1_ref[...] = jnp.dot(x, b_ref[...], preferred_element_type=jnp.float32)


def _tc_pre(x, A, B):
    bm = 2000
    return pl.pallas_call(
        _pre_body,
        grid=(_N // bm,),
        in_specs=[_rows(bm, _H), _full(_H, _H), _full(_H, _H)],
        out_specs=[_rows(bm, _H), _rows(bm, _H)],
        out_shape=[jax.ShapeDtypeStruct((_N, _H), jnp.float32)] * 2,
    )(x, A, B)


def _edge_body(xg0_ref, xg1_ref, ea_ref, c_ref, w1_ref, b0_ref, b1_ref,
               g_ref, bb_ref, o_ref):
    ea = ea_ref[...]
    t = (xg0_ref[...] + xg1_ref[...]
         + jnp.dot(ea, c_ref[...], preferred_element_type=jnp.float32)
         + b0_ref[...])
    t = _elu(t)
    u = jnp.dot(t, w1_ref[...], preferred_element_type=jnp.float32) + b1_ref[...]
    o_ref[...] = _ln(ea + u, g_ref[...], bb_ref[...])


def _tc_edge(xg0, xg1, ea, C, W1, b0, b1, g, bb):
    be = 8000
    e = ea.shape[0]
    return pl.pallas_call(
        _edge_body,
        grid=(e // be,),
        in_specs=[_rows(be, _H), _rows(be, _H), _rows(be, _H),
                  _full(_H, _H), _full(_H, _H),
                  _full(1, _H), _full(1, _H), _full(1, _H), _full(1, _H)],
        out_specs=_rows(be, _H),
        out_shape=jax.ShapeDtypeStruct((e, _H), jnp.float32),
    )(xg0, xg1, ea, C, W1, b0.reshape(1, _H), b1.reshape(1, _H),
      g.reshape(1, _H), bb.reshape(1, _H))


def _node_common(x, s0, s1, c, p_ref, q_ref, r_ref, b0_ref, b1_ref,
                 g_ref, bb_ref):
    cnt = jnp.maximum(c[:, 0:1], 1.0)
    agg = (s0 + s1) / cnt
    t = (jnp.dot(x, p_ref[...], preferred_element_type=jnp.float32)
         + jnp.dot(agg, q_ref[...], preferred_element_type=jnp.float32)
         + b0_ref[...])
    t = _elu(t)
    u = jnp.dot(t, r_ref[...], preferred_element_type=jnp.float32) + b1_ref[...]
    return _ln(x + u, g_ref[...], bb_ref[...])


def _node0_body(x_ref, s0_ref, s1_ref, c_ref,
                p_ref, q_ref, r_ref,
                b0_ref, b1_ref, g_ref, bb_ref, a_ref, b_ref,
                xo_ref, g0_ref, g1_ref):
    xn = _node_common(x_ref[...], s0_ref[...], s1_ref[...], c_ref[...],
                      p_ref, q_ref, r_ref, b0_ref, b1_ref,
                      g_ref, bb_ref)
    xo_ref[...] = xn
    g0_ref[...] = jnp.dot(xn, a_ref[...], preferred_element_type=jnp.float32)
    g1_ref[...] = jnp.dot(xn, b_ref[...], preferred_element_type=jnp.float32)


def _node1_body(x_ref, s0_ref, s1_ref, c_ref,
                p_ref, q_ref, r_ref,
                b0_ref, b1_ref, g_ref, bb_ref, ow_ref, ob_ref, o_ref):
    xn = _node_common(x_ref[...], s0_ref[...], s1_ref[...], c_ref[...],
                      p_ref, q_ref, r_ref, b0_ref, b1_ref,
                      g_ref, bb_ref)
    o_ref[...] = _elu(jnp.dot(xn, ow_ref[...],
                              preferred_element_type=jnp.float32) + ob_ref[...])


def _tc_node0(x, sa, c, P, Q, R, b0, b1, g, bb, A, B):
    bm = 2000
    return pl.pallas_call(
        _node0_body,
        grid=(_N // bm,),
        in_specs=[_rows(bm, _H)] * 4 +
                 [_full(_H, _H), _full(_H, _H), _full(_H, _H),
                  _full(1, _H), _full(1, _H), _full(1, _H), _full(1, _H),
                  _full(_H, _H), _full(_H, _H)],
        out_specs=[_rows(bm, _H)] * 3,
        out_shape=[jax.ShapeDtypeStruct((_N, _H), jnp.float32)] * 3,
    )(x, sa[0], sa[1], c, P, Q, R,
      b0.reshape(1, _H), b1.reshape(1, _H),
      g.reshape(1, _H), bb.reshape(1, _H), A, B)


def _tc_node1(x, sa, c, P, Q, R, b0, b1, g, bb, ow_pad, ob_pad):
    bm = 2000
    return pl.pallas_call(
        _node1_body,
        grid=(_N // bm,),
        in_specs=[_rows(bm, _H)] * 4 +
                 [_full(_H, _H), _full(_H, _H), _full(_H, _H),
                  _full(1, _H), _full(1, _H), _full(1, _H), _full(1, _H),
                  _full(_H, _H), _full(1, _H)],
        out_specs=_rows(bm, _H),
        out_shape=jax.ShapeDtypeStruct((_N, _H), jnp.float32),
    )(x, sa[0], sa[1], c, P, Q, R,
      b0.reshape(1, _H), b1.reshape(1, _H),
      g.reshape(1, _H), bb.reshape(1, _H), ow_pad, ob_pad)


@functools.lru_cache(maxsize=None)
def _sc_gather_kernel(e):
    epw = e // _NW
    cg = 80 if epw % 80 == 0 else 40
    nch = epw // cg

    def body(g0_hbm, g1_hbm, ei0_hbm, ei1_hbm, o0_hbm, o1_hbm,
             idx0_v, idx1_v, r0_v, r1_v, gsem, ws0, ws1, ws2, ws3):
        wsems = [ws0, ws1, ws2, ws3]
        wid = lax.axis_index("s") * _NC + lax.axis_index("c")
        base_w = wid * epw
        pltpu.sync_copy(ei0_hbm.at[pl.ds(base_w, epw)], idx0_v)
        pltpu.sync_copy(ei1_hbm.at[pl.ds(base_w, epw)], idx1_v)

        def chunk(k, b):
            off_l = k * cg
            return (pltpu.async_copy(g0_hbm.at[idx0_v.at[pl.ds(off_l, cg)]],
                                     r0_v.at[b], gsem),
                    pltpu.async_copy(g1_hbm.at[idx1_v.at[pl.ds(off_l, cg)]],
                                     r1_v.at[b], gsem))

        def wdrain(b):
            pltpu.make_async_copy(r0_v.at[b], o0_hbm.at[pl.ds(0, cg)],
                                  wsems[b]).wait()
            pltpu.make_async_copy(r1_v.at[b], o1_hbm.at[pl.ds(0, cg)],
                                  wsems[b]).wait()

        def group(j, carry):
            k0 = j * _GGRP
            cps = []
            for b in range(_GGRP):
                @pl.when(j > 0)
                def _d():
                    wdrain(b)
                cps.append(chunk(k0 + b, b))
            for b in range(_GGRP):
                off_g = base_w + (k0 + b) * cg
                cps[b][0].wait()
                cps[b][1].wait()
                pltpu.async_copy(r0_v.at[b], o0_hbm.at[pl.ds(off_g, cg)],
                                 wsems[b])
                pltpu.async_copy(r1_v.at[b], o1_hbm.at[pl.ds(off_g, cg)],
                                 wsems[b])
            return carry

        lax.fori_loop(0, nch // _GGRP, group, 0)
        for b in range(_GGRP):
            wdrain(b)
        for k in range(nch - nch % _GGRP, nch):
            cps = chunk(k, 0)
            off_g = base_w + k * cg
            cps[0].wait()
            cps[1].wait()
            pltpu.async_copy(r0_v.at[0], o0_hbm.at[pl.ds(off_g, cg)], ws0)
            pltpu.async_copy(r1_v.at[0], o1_hbm.at[pl.ds(off_g, cg)], ws0)
            wdrain(0)

    return pl.kernel(
        body,
        mesh=plsc.VectorSubcoreMesh(core_axis_name="c", subcore_axis_name="s"),
        out_type=[jax.ShapeDtypeStruct((e, _H), jnp.float32)] * 2,
        scratch_types=[
            pltpu.VMEM((epw,), jnp.int32),
            pltpu.VMEM((epw,), jnp.int32),
            pltpu.VMEM((_GGRP, cg, _H), jnp.float32),
            pltpu.VMEM((_GGRP, cg, _H), jnp.float32),
            pltpu.SemaphoreType.DMA,
            pltpu.SemaphoreType.DMA,
            pltpu.SemaphoreType.DMA,
            pltpu.SemaphoreType.DMA,
            pltpu.SemaphoreType.DMA,
        ],
    )


def _sc_gather(g0, g1, ei0, ei1):
    return _sc_gather_kernel(ei0.shape[0])(g0, g1, ei0, ei1)


@functools.lru_cache(maxsize=None)
def _sc_scatter_kernel(e):
    epw = e // _NW
    nch = epw // _SCG
    nsgrp = nch // _SGRP

    def body(ea_hbm, ei1_hbm, osum_hbm,
             idx_v, buf_v, stage_v, ssum_sh, gsem, isem,
             as0, as1, as2, as3):
        asems = [as0, as1, as2, as3]
        cid = lax.axis_index("c")
        sid = lax.axis_index("s")
        wid = sid * _NC + cid
        base_w = wid * epw
        zero16 = jnp.zeros((16,), jnp.float32)

        def zrow(r, c):
            for jj in range(_H // 16):
                stage_v[r, pl.ds(jj * 16, 16)] = zero16
            return c
        lax.fori_loop(0, _ST, zrow, 0)

        for t in range(_NST):
            pltpu.sync_copy(stage_v,
                            ssum_sh.at[pl.ds(sid * _SPR + t * _ST, _ST)])

        @pl.when(sid == 0)
        def _zero_tail():
            pltpu.sync_copy(stage_v.at[pl.ds(0, _TAIL)],
                            ssum_sh.at[pl.ds(_NS * _SPR, _TAIL)])

        plsc.subcore_barrier()

        def adrain(b):
            pltpu.make_async_copy(buf_v.at[b], ssum_sh.at[idx_v.at[b]],
                                  asems[b]).wait()

        def group(j, carry):
            base_g = base_w + j * _SGRP * _SCG
            ops = []
            for b in range(_SGRP):
                @pl.when(j > 0)
                def _d():
                    adrain(b)
                off = base_g + b * _SCG
                ops.append((pltpu.async_copy(ei1_hbm.at[pl.ds(off, _SCG)],
                                             idx_v.at[b], isem),
                            pltpu.async_copy(ea_hbm.at[pl.ds(off, _SCG)],
                                             buf_v.at[b], gsem)))
            for b in range(_SGRP):
                ops[b][0].wait()
                ops[b][1].wait()
                pltpu.async_copy(buf_v.at[b], ssum_sh.at[idx_v.at[b]],
                                 asems[b], add=True)
            return carry

        lax.fori_loop(0, nsgrp, group, 0)
        for b in range(_SGRP):
            adrain(b)
        for k in range(nch - nch % _SGRP, nch):
            off = base_w + k * _SCG
            pltpu.sync_copy(ei1_hbm.at[pl.ds(off, _SCG)], idx_v.at[0])
            pltpu.async_copy(ea_hbm.at[pl.ds(off, _SCG)],
                             buf_v.at[0], gsem).wait()
            pltpu.async_copy(buf_v.at[0], ssum_sh.at[idx_v.at[0]],
                             as0, add=True)
            adrain(0)
        plsc.subcore_barrier()

        for t in range(_NST):
            row0 = sid * _SPR + t * _ST
            pltpu.sync_copy(ssum_sh.at[pl.ds(row0, _ST)], stage_v)
            pltpu.sync_copy(stage_v, osum_hbm.at[cid, pl.ds(row0, _ST)])

        @pl.when(sid == 0)
        def _write_tail():
            pltpu.sync_copy(ssum_sh.at[pl.ds(_NS * _SPR, _TAIL)],
                            stage_v.at[pl.ds(0, _TAIL)])
            pltpu.sync_copy(stage_v.at[pl.ds(0, _TAIL)],
                            osum_hbm.at[cid, pl.ds(_NS * _SPR, _TAIL)])

    return pl.kernel(
        body,
        mesh=plsc.VectorSubcoreMesh(core_axis_name="c", subcore_axis_name="s"),
        out_type=jax.ShapeDtypeStruct((_NC, _N, _H), jnp.float32),
        scratch_types=[
            pltpu.VMEM((_SGRP, _SCG), jnp.int32),
            pltpu.VMEM((_SGRP, _SCG, _H), jnp.float32),
            pltpu.VMEM((_ST, _H), jnp.float32),
            pltpu.VMEM_SHARED((_N, _H), jnp.float32),
            pltpu.SemaphoreType.DMA,
            pltpu.SemaphoreType.DMA,
            pltpu.SemaphoreType.DMA,
            pltpu.SemaphoreType.DMA,
            pltpu.SemaphoreType.DMA,
            pltpu.SemaphoreType.DMA,
        ],
    )


def _sc_scatter(ea, ei1):
    return _sc_scatter_kernel(ea.shape[0])(ea, ei1)


_CST = 104


def _sc_count_body(ei1_hbm, ocnt_hbm, idx_v, ones_v, cstage_v, scnt_sh,
                   gsem, asem):
    cid = lax.axis_index("c")
    sid = lax.axis_index("s")
    wid = sid * _NC + cid
    base_w = wid * _EPW
    zero16 = jnp.zeros((16,), jnp.float32)
    one16 = jnp.full((16,), 1.0, jnp.float32)

    def zrow(r, c):
        for jj in range(_H // 16):
            cstage_v[r, pl.ds(jj * 16, 16)] = zero16
        return c
    lax.fori_loop(0, _CST, zrow, 0)

    def orow(r, c):
        for jj in range(_H // 16):
            ones_v[r, pl.ds(jj * 16, 16)] = one16
        return c
    lax.fori_loop(0, _CG, orow, 0)

    for t in range(_SPR // _CST):
        pltpu.sync_copy(cstage_v,
                        scnt_sh.at[pl.ds(sid * _SPR + t * _CST, _CST)])

    @pl.when(sid == 0)
    def _zero_tail():
        pltpu.sync_copy(cstage_v.at[pl.ds(0, _TAIL)],
                        scnt_sh.at[pl.ds(_NS * _SPR, _TAIL)])

    plsc.subcore_barrier()

    def group(j, carry):
        base_g = base_w + j * _GRP * _CG
        ics = []
        for b in range(_GRP):
            off = base_g + b * _CG
            ics.append(pltpu.async_copy(ei1_hbm.at[pl.ds(off, _CG)],
                                        idx_v.at[b], gsem))
        adds = []
        for b in range(_GRP):
            ics[b].wait()
            adds.append(pltpu.async_copy(ones_v, scnt_sh.at[idx_v.at[b]],
                                         asem, add=True))
        for a in adds:
            a.wait()
        return carry

    lax.fori_loop(0, _NGRP, group, 0)
    plsc.subcore_barrier()

    for t in range(_SPR // _CST):
        row0 = sid * _SPR + t * _CST
        pltpu.sync_copy(scnt_sh.at[pl.ds(row0, _CST)], cstage_v)
        pltpu.sync_copy(cstage_v, ocnt_hbm.at[cid, pl.ds(row0, _CST)])

    @pl.when(sid == 0)
    def _write_tail():
        pltpu.sync_copy(scnt_sh.at[pl.ds(_NS * _SPR, _TAIL)],
                        cstage_v.at[pl.ds(0, _TAIL)])
        pltpu.sync_copy(cstage_v.at[pl.ds(0, _TAIL)],
                        ocnt_hbm.at[cid, pl.ds(_NS * _SPR, _TAIL)])


@functools.lru_cache(maxsize=None)
def _sc_count_kernel():
    return pl.kernel(
        _sc_count_body,
        mesh=plsc.VectorSubcoreMesh(core_axis_name="c", subcore_axis_name="s"),
        out_type=jax.ShapeDtypeStruct((_NC, _N, _H), jnp.float32),
        scratch_types=[
            pltpu.VMEM((_GRP, _CG), jnp.int32),
            pltpu.VMEM((_CG, _H), jnp.float32),
            pltpu.VMEM((_CST, _H), jnp.float32),
            pltpu.VMEM_SHARED((_N, _H), jnp.float32),
            pltpu.SemaphoreType.DMA,
            pltpu.SemaphoreType.DMA,
        ],
    )


def _sc_count(ei1):
    return _sc_count_kernel()(ei1)


def _round(ea, ei0, ei1, g0, g1, C, W1, eb0, eb1, elg, elb):
    xg0, xg1 = _sc_gather(g0, g1, ei0, ei1)
    ea = _tc_edge(xg0, xg1, ea, C, W1, eb0, eb1, elg, elb)
    sums = _sc_scatter(ea, ei1)
    return ea, sums


def kernel(x, edge_index, edge_attr, edge_indices, edge_indices_f2c, clusters,
           batches, positions, lengthscales,
           ew00, eb00, ew01, eb01, elng0, elnb0, nw00, nb00, nw01, nb01,
           nlng0, nlnb0,
           ew10, eb10, ew11, eb11, elng1, elnb1, nw10, nb10, nw11, nb11,
           nlng1, nlnb1, ow, ob):
    ei = edge_indices[0]
    ei0 = ei[0]
    ei1 = ei[1]
    A0, B0, C0 = ew00[:_H], ew00[_H:2 * _H], ew00[2 * _H:]
    A1, B1, C1 = ew10[:_H], ew10[_H:2 * _H], ew10[2 * _H:]
    P0, Q0 = nw00[:_H], nw00[_H:]
    P1, Q1 = nw10[:_H], nw10[_H:]
    ow_pad = jnp.pad(ow, ((0, 0), (0, _H - _OUT)))
    ob_pad = jnp.pad(ob, (0, _H - _OUT)).reshape(1, _H)

    ocnt = _sc_count(ei1)
    cnt = ocnt[0] + ocnt[1]

    g0, g1 = _tc_pre(x, A0, B0)
    ea, sums_a = _round(edge_attr, ei0, ei1, g0, g1,
                        C0, ew01, eb00, eb01, elng0, elnb0)
    x, g0, g1 = _tc_node0(x, sums_a, cnt,
                          P0, Q0, nw01, nb00, nb01, nlng0, nlnb0, A1, B1)
    ea, sums_b = _round(ea, ei0, ei1, g0, g1,
                        C1, ew11, eb10, eb11, elng1, elnb1)
    out_pad = _tc_node1(x, sums_b, cnt,
                        P1, Q1, nw11, nb10, nb11, nlng1, nlnb1, ow_pad, ob_pad)
    return out_pad[:, :_OUT], ei
